# baseline (speedup 1.0000x reference)
import functools

import jax
import jax.numpy as jnp
from jax import lax
from jax.experimental import pallas as pl
from jax.experimental.pallas import tpu as pltpu
from jax.experimental.pallas import tpu_sc as plsc

_L_SEQ = 20
_VOCAB = 100000
_DIM = 32
_BATCH = 4096

_NSUB = 16
_G16 = _BATCH // 16

_mesh = plsc.VectorSubcoreMesh(core_axis_name="c", subcore_axis_name="s")


@functools.partial(
    pl.kernel,
    mesh=_mesh,
    out_type=jax.ShapeDtypeStruct((_L_SEQ, _DIM, _BATCH), jnp.float32),
    scratch_types=[
        pltpu.VMEM((_VOCAB,), jnp.float32),
        pltpu.VMEM((_BATCH,), jnp.int32),
        pltpu.VMEM((_BATCH,), jnp.float32),
        pltpu.SemaphoreType.DMA,
    ],
    compiler_params=pltpu.CompilerParams(
        use_tc_tiling_on_sc=True,
        needs_layout_passes=False,
        skip_device_barrier=True,
    ),
)
def _lookup_kernel(table_hbm, x_hbm, out_hbm, vocv, xv, orow, s_st):
    cc = lax.axis_index("c")
    sid = lax.axis_index("s")
    ch = cc * _NSUB + sid

    def stage_start(l):
        pltpu.async_copy(table_hbm.at[l * _DIM + ch, :], vocv, s_st)

    def stage_wait(l):
        pltpu.make_async_copy(
            table_hbm.at[l * _DIM + ch, :], vocv, s_st
        ).wait()

    stage_start(0)
    stage_wait(0)

    def round_body(l, carry):
        pltpu.sync_copy(x_hbm.at[l, :], xv)

        def gat(g, c):
            sl = pl.ds(g * 16, 16)
            orow[sl] = plsc.load_gather(vocv, [xv[sl]])
            return c

        lax.fori_loop(0, _G16, gat, 0, unroll=4)

        @pl.when(l < _L_SEQ - 1)
        def _():
            stage_start(l + 1)

        pltpu.sync_copy(orow, out_hbm.at[l, ch])

        @pl.when(l < _L_SEQ - 1)
        def _():
            stage_wait(l + 1)

        return carry

    lax.fori_loop(0, _L_SEQ, round_body, 0)


def kernel(x, tables):
    table_t = tables.transpose(0, 2, 1).reshape(_L_SEQ * _DIM, _VOCAB)
    x_t = x.T
    out_t = _lookup_kernel(table_t, x_t)
    return out_t.transpose(2, 0, 1)


# speedup vs baseline: 1.0886x; 1.0886x over previous
import functools

import jax
import jax.numpy as jnp
from jax import lax
from jax.experimental import pallas as pl
from jax.experimental.pallas import tpu as pltpu
from jax.experimental.pallas import tpu_sc as plsc

_L_SEQ = 20
_VOCAB = 100000
_DIM = 32
_BATCH = 4096

_NSUB = 16
_G16 = _BATCH // 16

_mesh = plsc.VectorSubcoreMesh(core_axis_name="c", subcore_axis_name="s")


@functools.partial(
    pl.kernel,
    mesh=_mesh,
    out_type=jax.ShapeDtypeStruct((_L_SEQ, _DIM, _BATCH), jnp.float32),
    scratch_types=[
        pltpu.VMEM((_VOCAB,), jnp.float32),
        pltpu.VMEM((_BATCH,), jnp.int32),
        pltpu.VMEM((_BATCH,), jnp.float32),
        pltpu.SemaphoreType.DMA,
    ],
    compiler_params=pltpu.CompilerParams(
        use_tc_tiling_on_sc=True,
        needs_layout_passes=False,
        skip_device_barrier=True,
    ),
)
def _lookup_kernel(table_hbm, x_hbm, out_hbm, vocv, xv, orow, s_st):
    cc = lax.axis_index("c")
    sid = lax.axis_index("s")
    ch = cc * _NSUB + sid

    def stage_start(l):
        pltpu.async_copy(table_hbm.at[l * _DIM + ch, :], vocv, s_st)

    def stage_wait(l):
        pltpu.make_async_copy(
            table_hbm.at[l * _DIM + ch, :], vocv, s_st
        ).wait()

    stage_start(0)
    stage_wait(0)

    def round_body(l, carry):
        pltpu.sync_copy(x_hbm.at[l, :], xv)

        def gat(g, c):
            sl = pl.ds(g * 16, 16)
            orow[sl] = plsc.load_gather(vocv, [xv[sl]])
            return c

        lax.fori_loop(0, _G16, gat, 0)

        @pl.when(l < _L_SEQ - 1)
        def _():
            stage_start(l + 1)

        pltpu.sync_copy(orow, out_hbm.at[l, ch])

        @pl.when(l < _L_SEQ - 1)
        def _():
            stage_wait(l + 1)

        return carry

    lax.fori_loop(0, _L_SEQ, round_body, 0)


def kernel(x, tables):
    table_t = tables.transpose(0, 2, 1).reshape(_L_SEQ * _DIM, _VOCAB)
    x_t = x.T
    out_t = _lookup_kernel(table_t, x_t)
    return out_t.transpose(2, 0, 1)
